# baseline (speedup 1.0000x reference)
import functools

import jax
import jax.numpy as jnp
from jax import lax
from jax.experimental import pallas as pl
from jax.experimental.pallas import tpu as pltpu
from jax.experimental.pallas import tpu_sc as plsc

B, C, H, W = 16, 256, 128, 128
HW = H * W
K = 128
HIDDEN = 1024
NUM_CLASSES = 117
LANES = 16
GRAN_ROWS = HW // LANES
N_PAIRS = K
N_WORKERS = 32


LAG = 4


def _sc_gather_kernel(table_hbm, inds_hbm, out_hbm, ind_v, idx_v, out_v, sem,
                      osem):
    wid = lax.axis_index("c") * 16 + lax.axis_index("s")
    b = lax.rem(wid, 16)
    base = b * (C * HW)
    pltpu.sync_copy(inds_hbm.at[pl.ds(wid * N_PAIRS, N_PAIRS)], ind_v)

    def fire(j, p):
        def chunk(c16, carry):
            cvec = (lax.iota(jnp.int32, LANES) + c16 * LANES) * HW
            idx_v[pl.ds(j * C + c16 * LANES, LANES)] = cvec + (p + base)
            return carry

        lax.fori_loop(0, C // LANES, chunk, None)
        pltpu.async_copy(table_hbm.at[idx_v.at[pl.ds(j * C, 128)]],
                         out_v.at[j, pl.ds(0, 128)], sem)
        pltpu.async_copy(table_hbm.at[idx_v.at[pl.ds(j * C + 128, 128)]],
                         out_v.at[j, pl.ds(128, 128)], sem)

    def drain(j):
        pltpu.make_async_copy(table_hbm.at[pl.ds(0, C)],
                              out_v.at[j], sem).wait()

    def write_group(g):
        pltpu.async_copy(
            out_v.at[pl.ds(g * LANES, LANES), :],
            out_hbm.at[pl.ds(wid * N_PAIRS + g * LANES, LANES), :], osem)

    def group_body(g, carry):
        p_vec = ind_v[pl.ds(g * LANES, LANES)]
        for j16 in range(LANES):
            fire(g * LANES + j16, p_vec[j16])

        @pl.when(g >= 1)
        def _():
            for j16 in range(LANES):
                drain((g - 1) * LANES + j16)
            write_group(g - 1)

        return carry

    n_groups = N_PAIRS // LANES
    lax.fori_loop(0, n_groups, group_body, None)
    for j in range(N_PAIRS - LANES, N_PAIRS):
        drain(j)
    write_group(n_groups - 1)
    for _ in range(n_groups):
        pltpu.make_async_copy(
            out_v.at[pl.ds(0, LANES), :],
            out_hbm.at[pl.ds(wid * N_PAIRS, LANES), :], osem).wait()


def _mlp_body(sub_ref, obj_ref, w1a_ref, w1b_ref, b1_ref, w2_ref, b2_ref,
              w3_ref, b3_ref, out_ref):
    dn = (((1,), (1,)), ((), ()))
    bf = jnp.bfloat16
    x = lax.dot_general(sub_ref[...].astype(bf), w1a_ref[...], dn,
                        preferred_element_type=jnp.float32)
    x = x + lax.dot_general(obj_ref[...].astype(bf), w1b_ref[...], dn,
                            preferred_element_type=jnp.float32)
    h1 = jnp.maximum(x + b1_ref[...], 0.0).astype(bf)
    h2 = jnp.maximum(
        lax.dot_general(h1, w2_ref[...], dn,
                        preferred_element_type=jnp.float32) + b2_ref[...],
        0.0).astype(bf)
    out = lax.dot_general(
        h2, w3_ref[...], dn, preferred_element_type=jnp.float32) + b3_ref[...]
    out_ref[...] = out.reshape(out_ref.shape)


def kernel(embedding, sub_ind, obj_ind, W1, b1, W2, b2, W3, b3):
    table = embedding.reshape(B * C * HW)
    inds = jnp.concatenate(
        [sub_ind.reshape(-1).astype(jnp.int32),
         obj_ind.reshape(-1).astype(jnp.int32)])

    mesh = plsc.VectorSubcoreMesh(core_axis_name="c", subcore_axis_name="s")
    gathered = pl.kernel(
        _sc_gather_kernel,
        mesh=mesh,
        out_type=jax.ShapeDtypeStruct((2 * B * K, C), jnp.float32),
        scratch_types=[
            pltpu.VMEM((N_PAIRS,), jnp.int32),
            pltpu.VMEM((N_PAIRS * C,), jnp.int32),
            pltpu.VMEM((N_PAIRS, C), jnp.float32),
            pltpu.SemaphoreType.DMA,
            pltpu.SemaphoreType.DMA,
        ],
    )(table, inds)

    feats = gathered
    n_sub_blocks = B * K // 512

    m_tile = 512
    out = pl.pallas_call(
        _mlp_body,
        grid=(B * K // m_tile,),
        in_specs=[
            pl.BlockSpec((m_tile, C), lambda m: (m, 0)),
            pl.BlockSpec((m_tile, C), lambda m: (m + n_sub_blocks, 0)),
            pl.BlockSpec((HIDDEN, C), lambda m: (0, 0)),
            pl.BlockSpec((HIDDEN, C), lambda m: (0, 1)),
            pl.BlockSpec((1, HIDDEN), lambda m: (0, 0)),
            pl.BlockSpec((HIDDEN, HIDDEN), lambda m: (0, 0)),
            pl.BlockSpec((1, HIDDEN), lambda m: (0, 0)),
            pl.BlockSpec((NUM_CLASSES, HIDDEN), lambda m: (0, 0)),
            pl.BlockSpec((1, NUM_CLASSES), lambda m: (0, 0)),
        ],
        out_specs=pl.BlockSpec((m_tile // K, K, NUM_CLASSES),
                               lambda m: (m, 0, 0)),
        out_shape=jax.ShapeDtypeStruct((B, K, NUM_CLASSES), jnp.float32),
        compiler_params=pltpu.CompilerParams(
            dimension_semantics=("parallel",)),
    )(feats, feats, W1.astype(jnp.bfloat16), W1.astype(jnp.bfloat16),
      b1.reshape(1, HIDDEN), W2.astype(jnp.bfloat16),
      b2.reshape(1, HIDDEN), W3.astype(jnp.bfloat16),
      b3.reshape(1, NUM_CLASSES))

    return out


# speedup vs baseline: 1.0231x; 1.0231x over previous
import functools

import jax
import jax.numpy as jnp
from jax import lax
from jax.experimental import pallas as pl
from jax.experimental.pallas import tpu as pltpu
from jax.experimental.pallas import tpu_sc as plsc

B, C, H, W = 16, 256, 128, 128
HW = H * W
K = 128
HIDDEN = 1024
NUM_CLASSES = 117
LANES = 16
GRAN_ROWS = HW // LANES
N_PAIRS = K
N_WORKERS = 32


LAG = 4


def _sc_gather_kernel(table_hbm, inds_hbm, out_hbm, ind_v, idx_v, out_v, sem,
                      osem):
    wid = lax.axis_index("c") * 16 + lax.axis_index("s")
    b = lax.rem(wid, 16)
    base = b * (C * HW)
    pltpu.sync_copy(inds_hbm.at[pl.ds(wid * N_PAIRS, N_PAIRS)], ind_v)

    def fire(j, p):
        def chunk(c16, carry):
            cvec = (lax.iota(jnp.int32, LANES) + c16 * LANES) * HW
            idx_v[pl.ds(j * C + c16 * LANES, LANES)] = cvec + (p + base)
            return carry

        lax.fori_loop(0, C // LANES, chunk, None)
        pltpu.async_copy(table_hbm.at[idx_v.at[pl.ds(j * C, 128)]],
                         out_v.at[j, pl.ds(0, 128)], sem)
        pltpu.async_copy(table_hbm.at[idx_v.at[pl.ds(j * C + 128, 128)]],
                         out_v.at[j, pl.ds(128, 128)], sem)

    def drain(j):
        pltpu.make_async_copy(table_hbm.at[pl.ds(0, C)],
                              out_v.at[j], sem).wait()

    def write_group(g):
        pltpu.async_copy(
            out_v.at[pl.ds(g * LANES, LANES), :],
            out_hbm.at[pl.ds(wid * N_PAIRS + g * LANES, LANES), :], osem)

    def group_body(g, carry):
        p_vec = ind_v[pl.ds(g * LANES, LANES)]
        for j16 in range(LANES):
            fire(g * LANES + j16, p_vec[j16])

        @pl.when(g >= 2)
        def _():
            for j16 in range(LANES):
                drain((g - 2) * LANES + j16)
            write_group(g - 2)

        return carry

    n_groups = N_PAIRS // LANES
    lax.fori_loop(0, n_groups, group_body, None)
    for j in range((n_groups - 2) * LANES, N_PAIRS):
        drain(j)
    write_group(n_groups - 2)
    write_group(n_groups - 1)
    for _ in range(n_groups):
        pltpu.make_async_copy(
            out_v.at[pl.ds(0, LANES), :],
            out_hbm.at[pl.ds(wid * N_PAIRS, LANES), :], osem).wait()


def _mlp_body(sub_ref, obj_ref, w1a_ref, w1b_ref, b1_ref, w2_ref, b2_ref,
              w3_ref, b3_ref, out_ref):
    dn = (((1,), (1,)), ((), ()))
    bf = jnp.bfloat16
    x = lax.dot_general(sub_ref[...].astype(bf), w1a_ref[...], dn,
                        preferred_element_type=jnp.float32)
    x = x + lax.dot_general(obj_ref[...].astype(bf), w1b_ref[...], dn,
                            preferred_element_type=jnp.float32)
    h1 = jnp.maximum(x + b1_ref[...], 0.0).astype(bf)
    h2 = jnp.maximum(
        lax.dot_general(h1, w2_ref[...], dn,
                        preferred_element_type=jnp.float32) + b2_ref[...],
        0.0).astype(bf)
    out = lax.dot_general(
        h2, w3_ref[...], dn, preferred_element_type=jnp.float32) + b3_ref[...]
    out_ref[...] = out.reshape(out_ref.shape)


def kernel(embedding, sub_ind, obj_ind, W1, b1, W2, b2, W3, b3):
    table = embedding.reshape(B * C * HW)
    inds = jnp.concatenate(
        [sub_ind.reshape(-1).astype(jnp.int32),
         obj_ind.reshape(-1).astype(jnp.int32)])

    mesh = plsc.VectorSubcoreMesh(core_axis_name="c", subcore_axis_name="s")
    gathered = pl.kernel(
        _sc_gather_kernel,
        mesh=mesh,
        out_type=jax.ShapeDtypeStruct((2 * B * K, C), jnp.float32),
        scratch_types=[
            pltpu.VMEM((N_PAIRS,), jnp.int32),
            pltpu.VMEM((N_PAIRS * C,), jnp.int32),
            pltpu.VMEM((N_PAIRS, C), jnp.float32),
            pltpu.SemaphoreType.DMA,
            pltpu.SemaphoreType.DMA,
        ],
    )(table, inds)

    feats = gathered
    n_sub_blocks = B * K // 512

    m_tile = 512
    out = pl.pallas_call(
        _mlp_body,
        grid=(B * K // m_tile,),
        in_specs=[
            pl.BlockSpec((m_tile, C), lambda m: (m, 0)),
            pl.BlockSpec((m_tile, C), lambda m: (m + n_sub_blocks, 0)),
            pl.BlockSpec((HIDDEN, C), lambda m: (0, 0)),
            pl.BlockSpec((HIDDEN, C), lambda m: (0, 1)),
            pl.BlockSpec((1, HIDDEN), lambda m: (0, 0)),
            pl.BlockSpec((HIDDEN, HIDDEN), lambda m: (0, 0)),
            pl.BlockSpec((1, HIDDEN), lambda m: (0, 0)),
            pl.BlockSpec((NUM_CLASSES, HIDDEN), lambda m: (0, 0)),
            pl.BlockSpec((1, NUM_CLASSES), lambda m: (0, 0)),
        ],
        out_specs=pl.BlockSpec((m_tile // K, K, NUM_CLASSES),
                               lambda m: (m, 0, 0)),
        out_shape=jax.ShapeDtypeStruct((B, K, NUM_CLASSES), jnp.float32),
        compiler_params=pltpu.CompilerParams(
            dimension_semantics=("parallel",)),
    )(feats, feats, W1.astype(jnp.bfloat16), W1.astype(jnp.bfloat16),
      b1.reshape(1, HIDDEN), W2.astype(jnp.bfloat16),
      b2.reshape(1, HIDDEN), W3.astype(jnp.bfloat16),
      b3.reshape(1, NUM_CLASSES))

    return out


# speedup vs baseline: 1.0311x; 1.0079x over previous
import functools

import jax
import jax.numpy as jnp
from jax import lax
from jax.experimental import pallas as pl
from jax.experimental.pallas import tpu as pltpu
from jax.experimental.pallas import tpu_sc as plsc

B, C, H, W = 16, 256, 128, 128
HW = H * W
K = 128
HIDDEN = 1024
NUM_CLASSES = 117
LANES = 16
GRAN_ROWS = HW // LANES
N_PAIRS = K
N_WORKERS = 32


LAG = 4


def _sc_gather_kernel(table_hbm, inds_hbm, out_hbm, ind_v, idx_v, out_v, sem,
                      osem):
    wid = lax.axis_index("c") * 16 + lax.axis_index("s")
    b = lax.rem(wid, 16)
    base = b * (C * HW)
    pltpu.sync_copy(inds_hbm.at[pl.ds(wid * N_PAIRS, N_PAIRS)], ind_v)

    def fire(j, p):
        def chunk(c16, carry):
            cvec = (lax.iota(jnp.int32, LANES) + c16 * LANES) * HW
            idx_v[pl.ds(j * C + c16 * LANES, LANES)] = cvec + (p + base)
            return carry

        lax.fori_loop(0, C // LANES, chunk, None)
        pltpu.async_copy(table_hbm.at[idx_v.at[pl.ds(j * C, 128)]],
                         out_v.at[j, pl.ds(0, 128)], sem)
        pltpu.async_copy(table_hbm.at[idx_v.at[pl.ds(j * C + 128, 128)]],
                         out_v.at[j, pl.ds(128, 128)], sem)

    def drain(j):
        pltpu.make_async_copy(table_hbm.at[pl.ds(0, C)],
                              out_v.at[j], sem).wait()

    def write_group(g):
        pltpu.async_copy(
            out_v.at[pl.ds(g * LANES, LANES), :],
            out_hbm.at[pl.ds(wid * N_PAIRS + g * LANES, LANES), :], osem)

    def group_body(g, carry):
        p_vec = ind_v[pl.ds(g * LANES, LANES)]
        for j16 in range(LANES):
            fire(g * LANES + j16, p_vec[j16])

        @pl.when(g >= 3)
        def _():
            for j16 in range(LANES):
                drain((g - 3) * LANES + j16)
            write_group(g - 3)

        return carry

    n_groups = N_PAIRS // LANES
    lax.fori_loop(0, n_groups, group_body, None)
    for j in range((n_groups - 3) * LANES, N_PAIRS):
        drain(j)
    write_group(n_groups - 3)
    write_group(n_groups - 2)
    write_group(n_groups - 1)
    for _ in range(n_groups):
        pltpu.make_async_copy(
            out_v.at[pl.ds(0, LANES), :],
            out_hbm.at[pl.ds(wid * N_PAIRS, LANES), :], osem).wait()


def _mlp_body(sub_ref, obj_ref, w1a_ref, w1b_ref, b1_ref, w2_ref, b2_ref,
              w3_ref, b3_ref, out_ref):
    dn = (((1,), (1,)), ((), ()))
    bf = jnp.bfloat16
    x = lax.dot_general(sub_ref[...].astype(bf), w1a_ref[...], dn,
                        preferred_element_type=jnp.float32)
    x = x + lax.dot_general(obj_ref[...].astype(bf), w1b_ref[...], dn,
                            preferred_element_type=jnp.float32)
    h1 = jnp.maximum(x + b1_ref[...], 0.0).astype(bf)
    h2 = jnp.maximum(
        lax.dot_general(h1, w2_ref[...], dn,
                        preferred_element_type=jnp.float32) + b2_ref[...],
        0.0).astype(bf)
    out = lax.dot_general(
        h2, w3_ref[...], dn, preferred_element_type=jnp.float32) + b3_ref[...]
    out_ref[...] = out.reshape(out_ref.shape)


def kernel(embedding, sub_ind, obj_ind, W1, b1, W2, b2, W3, b3):
    table = embedding.reshape(B * C * HW)
    inds = jnp.concatenate(
        [sub_ind.reshape(-1).astype(jnp.int32),
         obj_ind.reshape(-1).astype(jnp.int32)])

    mesh = plsc.VectorSubcoreMesh(core_axis_name="c", subcore_axis_name="s")
    gathered = pl.kernel(
        _sc_gather_kernel,
        mesh=mesh,
        out_type=jax.ShapeDtypeStruct((2 * B * K, C), jnp.float32),
        scratch_types=[
            pltpu.VMEM((N_PAIRS,), jnp.int32),
            pltpu.VMEM((N_PAIRS * C,), jnp.int32),
            pltpu.VMEM((N_PAIRS, C), jnp.float32),
            pltpu.SemaphoreType.DMA,
            pltpu.SemaphoreType.DMA,
        ],
    )(table, inds)

    feats = gathered
    n_sub_blocks = B * K // 512

    m_tile = 512
    out = pl.pallas_call(
        _mlp_body,
        grid=(B * K // m_tile,),
        in_specs=[
            pl.BlockSpec((m_tile, C), lambda m: (m, 0)),
            pl.BlockSpec((m_tile, C), lambda m: (m + n_sub_blocks, 0)),
            pl.BlockSpec((HIDDEN, C), lambda m: (0, 0)),
            pl.BlockSpec((HIDDEN, C), lambda m: (0, 1)),
            pl.BlockSpec((1, HIDDEN), lambda m: (0, 0)),
            pl.BlockSpec((HIDDEN, HIDDEN), lambda m: (0, 0)),
            pl.BlockSpec((1, HIDDEN), lambda m: (0, 0)),
            pl.BlockSpec((NUM_CLASSES, HIDDEN), lambda m: (0, 0)),
            pl.BlockSpec((1, NUM_CLASSES), lambda m: (0, 0)),
        ],
        out_specs=pl.BlockSpec((m_tile // K, K, NUM_CLASSES),
                               lambda m: (m, 0, 0)),
        out_shape=jax.ShapeDtypeStruct((B, K, NUM_CLASSES), jnp.float32),
        compiler_params=pltpu.CompilerParams(
            dimension_semantics=("parallel",)),
    )(feats, feats, W1.astype(jnp.bfloat16), W1.astype(jnp.bfloat16),
      b1.reshape(1, HIDDEN), W2.astype(jnp.bfloat16),
      b2.reshape(1, HIDDEN), W3.astype(jnp.bfloat16),
      b3.reshape(1, NUM_CLASSES))

    return out


# speedup vs baseline: 1.0366x; 1.0053x over previous
import functools

import jax
import jax.numpy as jnp
from jax import lax
from jax.experimental import pallas as pl
from jax.experimental.pallas import tpu as pltpu
from jax.experimental.pallas import tpu_sc as plsc

B, C, H, W = 16, 256, 128, 128
HW = H * W
K = 128
HIDDEN = 1024
NUM_CLASSES = 117
LANES = 16
GRAN_ROWS = HW // LANES
N_PAIRS = K
N_WORKERS = 32


LAG = 4


def _sc_gather_kernel(table_hbm, inds_hbm, out_hbm, ind_v, idx_v, out_v, sem,
                      osem):
    wid = lax.axis_index("c") * 16 + lax.axis_index("s")
    b = lax.rem(wid, 16)
    base = b * (C * HW)
    pltpu.sync_copy(inds_hbm.at[pl.ds(wid * N_PAIRS, N_PAIRS)], ind_v)

    def fire(j, p):
        def chunk(c16, carry):
            cvec = (lax.iota(jnp.int32, LANES) + c16 * LANES) * HW
            idx_v[pl.ds(j * C + c16 * LANES, LANES)] = cvec + (p + base)
            return carry

        lax.fori_loop(0, C // LANES, chunk, None)
        pltpu.async_copy(table_hbm.at[idx_v.at[pl.ds(j * C, 128)]],
                         out_v.at[j, pl.ds(0, 128)], sem)
        pltpu.async_copy(table_hbm.at[idx_v.at[pl.ds(j * C + 128, 128)]],
                         out_v.at[j, pl.ds(128, 128)], sem)

    def drain(j):
        pltpu.make_async_copy(table_hbm.at[pl.ds(0, C)],
                              out_v.at[j], sem).wait()

    def write_group(g):
        pltpu.async_copy(
            out_v.at[pl.ds(g * LANES, LANES), :],
            out_hbm.at[pl.ds(wid * N_PAIRS + g * LANES, LANES), :], osem)

    def group_body(g, carry):
        p_vec = ind_v[pl.ds(g * LANES, LANES)]
        for j16 in range(LANES):
            fire(g * LANES + j16, p_vec[j16])

        @pl.when(g >= 5)
        def _():
            for j16 in range(LANES):
                drain((g - 5) * LANES + j16)
            write_group(g - 5)

        return carry

    n_groups = N_PAIRS // LANES
    lax.fori_loop(0, n_groups, group_body, None)
    for j in range((n_groups - 5) * LANES, N_PAIRS):
        drain(j)
    for g in range(n_groups - 5, n_groups):
        write_group(g)
    for _ in range(n_groups):
        pltpu.make_async_copy(
            out_v.at[pl.ds(0, LANES), :],
            out_hbm.at[pl.ds(wid * N_PAIRS, LANES), :], osem).wait()


def _mlp_body(sub_ref, obj_ref, w1a_ref, w1b_ref, b1_ref, w2_ref, b2_ref,
              w3_ref, b3_ref, out_ref):
    dn = (((1,), (1,)), ((), ()))
    bf = jnp.bfloat16
    x = lax.dot_general(sub_ref[...].astype(bf), w1a_ref[...], dn,
                        preferred_element_type=jnp.float32)
    x = x + lax.dot_general(obj_ref[...].astype(bf), w1b_ref[...], dn,
                            preferred_element_type=jnp.float32)
    h1 = jnp.maximum(x + b1_ref[...], 0.0).astype(bf)
    h2 = jnp.maximum(
        lax.dot_general(h1, w2_ref[...], dn,
                        preferred_element_type=jnp.float32) + b2_ref[...],
        0.0).astype(bf)
    out = lax.dot_general(
        h2, w3_ref[...], dn, preferred_element_type=jnp.float32) + b3_ref[...]
    out_ref[...] = out.reshape(out_ref.shape)


def kernel(embedding, sub_ind, obj_ind, W1, b1, W2, b2, W3, b3):
    table = embedding.reshape(B * C * HW)
    inds = jnp.concatenate(
        [sub_ind.reshape(-1).astype(jnp.int32),
         obj_ind.reshape(-1).astype(jnp.int32)])

    mesh = plsc.VectorSubcoreMesh(core_axis_name="c", subcore_axis_name="s")
    gathered = pl.kernel(
        _sc_gather_kernel,
        mesh=mesh,
        out_type=jax.ShapeDtypeStruct((2 * B * K, C), jnp.float32),
        scratch_types=[
            pltpu.VMEM((N_PAIRS,), jnp.int32),
            pltpu.VMEM((N_PAIRS * C,), jnp.int32),
            pltpu.VMEM((N_PAIRS, C), jnp.float32),
            pltpu.SemaphoreType.DMA,
            pltpu.SemaphoreType.DMA,
        ],
    )(table, inds)

    feats = gathered
    n_sub_blocks = B * K // 512

    m_tile = 512
    out = pl.pallas_call(
        _mlp_body,
        grid=(B * K // m_tile,),
        in_specs=[
            pl.BlockSpec((m_tile, C), lambda m: (m, 0)),
            pl.BlockSpec((m_tile, C), lambda m: (m + n_sub_blocks, 0)),
            pl.BlockSpec((HIDDEN, C), lambda m: (0, 0)),
            pl.BlockSpec((HIDDEN, C), lambda m: (0, 1)),
            pl.BlockSpec((1, HIDDEN), lambda m: (0, 0)),
            pl.BlockSpec((HIDDEN, HIDDEN), lambda m: (0, 0)),
            pl.BlockSpec((1, HIDDEN), lambda m: (0, 0)),
            pl.BlockSpec((NUM_CLASSES, HIDDEN), lambda m: (0, 0)),
            pl.BlockSpec((1, NUM_CLASSES), lambda m: (0, 0)),
        ],
        out_specs=pl.BlockSpec((m_tile // K, K, NUM_CLASSES),
                               lambda m: (m, 0, 0)),
        out_shape=jax.ShapeDtypeStruct((B, K, NUM_CLASSES), jnp.float32),
        compiler_params=pltpu.CompilerParams(
            dimension_semantics=("parallel",)),
    )(feats, feats, W1.astype(jnp.bfloat16), W1.astype(jnp.bfloat16),
      b1.reshape(1, HIDDEN), W2.astype(jnp.bfloat16),
      b2.reshape(1, HIDDEN), W3.astype(jnp.bfloat16),
      b3.reshape(1, NUM_CLASSES))

    return out
